# Initial kernel scaffold; baseline (speedup 1.0000x reference)
import jax
import jax.numpy as jnp
from jax.experimental import pallas as pl


def kernel(h, edge_index, edge_weight, W1, b1, W2, b2, fcW1, fcb1, fcW2, fcb2, bn1_gamma, bn1_beta, bn2_gamma, bn2_beta):
    raise NotImplementedError("write your pallas kernel here")


# speedup vs baseline: 4.2701x; 4.2701x over previous
import functools

import jax
import jax.numpy as jnp
from jax import lax
from jax.experimental import pallas as pl
from jax.experimental.pallas import tpu as pltpu
from jax.experimental.pallas import tpu_sc as plsc

NC = 2
NS = 16
NW = NC * NS


def _rsqrt16(x):
    i = lax.bitcast_convert_type(x, jnp.int32)
    i = jnp.int32(0x5F3759DF) - lax.shift_right_logical(i, 1)
    y = lax.bitcast_convert_type(i, jnp.float32)
    for _ in range(3):
        y = y * (1.5 - 0.5 * x * y * y)
    return y


@functools.lru_cache(maxsize=None)
def _make_deg_kernel(N, E, Np):
    per_tile = E // NS
    CH = 2000
    assert per_tile % CH == 0
    nwin = per_tile // CH
    SL = Np // NS
    assert SL % 16 == 0 and SL % 8 == 0
    mesh = plsc.VectorSubcoreMesh(core_axis_name="c", subcore_axis_name="s")

    @functools.partial(
        pl.kernel,
        out_type=(jax.ShapeDtypeStruct((Np,), jnp.float32),
                  jax.ShapeDtypeStruct((Np,), jnp.float32)),
        mesh=mesh,
        scratch_types=[
            pltpu.VMEM((CH,), jnp.int32),
            pltpu.VMEM((CH,), jnp.float32),
            pltpu.VMEM((SL,), jnp.float32),
            pltpu.VMEM_SHARED((Np,), jnp.float32),
        ],
        compiler_params=pltpu.CompilerParams(needs_layout_passes=False),
    )
    def deg_k(src_hbm, dst_hbm, sout_hbm, sin_hbm, idx_v, ones_v, slc_v,
              cnt_sh):
        c = lax.axis_index("c")
        s = lax.axis_index("s")
        for j in range(SL // 16):
            slc_v[pl.ds(16 * j, 16)] = jnp.zeros((16,), jnp.float32)
        pltpu.sync_copy(slc_v, cnt_sh.at[pl.ds(s * SL, SL)])
        for j in range(CH // 16):
            ones_v[pl.ds(16 * j, 16)] = jnp.ones((16,), jnp.float32)
        plsc.subcore_barrier()

        def win_body(w, carry):
            base = s * per_tile + w * CH

            @pl.when(c == 0)
            def _():
                pltpu.sync_copy(src_hbm.at[pl.ds(base, CH)], idx_v)

            @pl.when(c == 1)
            def _():
                pltpu.sync_copy(dst_hbm.at[pl.ds(base, CH)], idx_v)

            pltpu.sync_copy(ones_v, cnt_sh.at[idx_v], add=True)
            return carry

        lax.fori_loop(0, nwin, win_body, 0)
        plsc.subcore_barrier()

        pltpu.sync_copy(cnt_sh.at[pl.ds(s * SL, SL)], slc_v)
        for j in range(SL // 16):
            x = jnp.maximum(slc_v[pl.ds(16 * j, 16)], 1.0)
            slc_v[pl.ds(16 * j, 16)] = _rsqrt16(x)

        @pl.when(c == 0)
        def _():
            pltpu.sync_copy(slc_v, sout_hbm.at[pl.ds(s * SL, SL)])

        @pl.when(c == 1)
        def _():
            pltpu.sync_copy(slc_v, sin_hbm.at[pl.ds(s * SL, SL)])

    return deg_k


@functools.lru_cache(maxsize=None)
def _make_agg_kernel(N, E, D, Np):
    EPW = E // NW
    K = 288
    assert EPW % K == 0
    NWIN = EPW // K
    RSTEP = (N // NS) & ~7
    RPT = N - RSTEP * (NS - 1)
    assert RPT % 8 == 0 and RPT >= RSTEP
    mesh = plsc.VectorSubcoreMesh(core_axis_name="c", subcore_axis_name="s")

    @functools.partial(
        pl.kernel,
        out_type=jax.ShapeDtypeStruct((NC, N, D), jnp.float32),
        mesh=mesh,
        scratch_types=[
            pltpu.VMEM((K,), jnp.int32),
            pltpu.VMEM((K,), jnp.int32),
            pltpu.VMEM((K,), jnp.float32),
            pltpu.VMEM((K,), jnp.float32),
            pltpu.VMEM((K, D), jnp.float32),
            pltpu.VMEM((Np,), jnp.float32),
            pltpu.VMEM_SHARED((N, D), jnp.float32),
            pltpu.SemaphoreType.DMA,
        ],
        compiler_params=pltpu.CompilerParams(needs_layout_passes=False),
    )
    def agg_k(x_hbm, src_hbm, dst_hbm, ew_hbm, sout_hbm, out_hbm,
              is_v, id_v, ew_v, w_v, rows_v, tab_v, agg_sh, sem):
        c = lax.axis_index("c")
        s = lax.axis_index("s")
        wid = c * NS + s
        pltpu.sync_copy(sout_hbm, tab_v)

        def zr(i, carry):
            for j in range(D // 16):
                rows_v[i, pl.ds(16 * j, 16)] = jnp.zeros((16,), jnp.float32)
            return carry

        lax.fori_loop(0, K, zr, 0)
        done = 0
        while done < RPT:
            step = min(K, RPT - done)
            pltpu.sync_copy(rows_v.at[pl.ds(0, step), :],
                            agg_sh.at[pl.ds(s * RSTEP + done, step), :])
            done += step
        plsc.subcore_barrier()

        def win_body(wi, carry):
            e0 = wid * EPW + wi * K
            pltpu.sync_copy(src_hbm.at[pl.ds(e0, K)], is_v)
            pltpu.sync_copy(dst_hbm.at[pl.ds(e0, K)], id_v)
            pltpu.sync_copy(ew_hbm.at[pl.ds(e0, K)], ew_v)

            def wcomp(i, c2):
                iv = is_v[pl.ds(i * 16, 16)]
                sov = plsc.load_gather(tab_v, [iv])
                w_v[pl.ds(i * 16, 16)] = ew_v[pl.ds(i * 16, 16)] * sov
                return c2

            lax.fori_loop(0, K // 16, wcomp, 0)
            pltpu.async_copy(x_hbm.at[is_v], rows_v, sem).wait()

            def scale(e, c2):
                wv = plsc.load_gather(w_v, [jnp.full((16,), e, jnp.int32)])
                for j in range(D // 16):
                    rows_v[e, pl.ds(16 * j, 16)] = (
                        rows_v[e, pl.ds(16 * j, 16)] * wv)
                return c2

            lax.fori_loop(0, K, scale, 0)
            pltpu.sync_copy(rows_v, agg_sh.at[id_v], add=True)
            return carry

        lax.fori_loop(0, NWIN, win_body, 0)
        plsc.subcore_barrier()
        pltpu.sync_copy(agg_sh.at[pl.ds(s * RSTEP, RPT), :],
                        out_hbm.at[c, pl.ds(s * RSTEP, RPT), :])

    return agg_k


_BN_INV = 0.9999950000374997


def _dense_body(p0_r, p1_r, s_r, W_r, b_r, fW_r, fb_r, g_r, be_r, o_r):
    a = (p0_r[...] + p1_r[...]) * s_r[...]
    t = jnp.dot(a, W_r[...], preferred_element_type=jnp.float32) + b_r[...]
    t = jnp.maximum(t, 0.0)
    t = jnp.dot(t, fW_r[...], preferred_element_type=jnp.float32) + fb_r[...]
    t = jnp.maximum(t, 0.0)
    o_r[...] = t * (g_r[...] * _BN_INV) + be_r[...]


def _dense_mean_body(nblocks, n_total, p0_r, p1_r, s_r, W_r, b_r, fW_r, fb_r,
                     g_r, be_r, o_r):
    i = pl.program_id(0)
    a = (p0_r[...] + p1_r[...]) * s_r[...]
    t = jnp.dot(a, W_r[...], preferred_element_type=jnp.float32) + b_r[...]
    t = jnp.maximum(t, 0.0)
    t = jnp.dot(t, fW_r[...], preferred_element_type=jnp.float32) + fb_r[...]
    t = jnp.maximum(t, 0.0)
    x = t * (g_r[...] * _BN_INV) + be_r[...]
    part = jnp.sum(x, axis=0, keepdims=True) * (1.0 / n_total)

    @pl.when(i == 0)
    def _():
        o_r[...] = part

    @pl.when(i > 0)
    def _():
        o_r[...] = o_r[...] + part


def _tc_dense(p0, p1, s_col, W, b, fW, fb, g, be, mean):
    N, D = p0.shape
    R = 1000
    assert N % R == 0
    grid = (N // R,)
    full = lambda i: (0, 0)
    blk = lambda i: (i, 0)
    in_specs = [
        pl.BlockSpec((R, D), blk),
        pl.BlockSpec((R, D), blk),
        pl.BlockSpec((R, 1), blk),
        pl.BlockSpec((D, D), full),
        pl.BlockSpec((1, D), full),
        pl.BlockSpec((D, D), full),
        pl.BlockSpec((1, D), full),
        pl.BlockSpec((1, D), full),
        pl.BlockSpec((1, D), full),
    ]
    if mean:
        body = functools.partial(_dense_mean_body, N // R, N)
        out_specs = pl.BlockSpec((1, D), full)
        out_shape = jax.ShapeDtypeStruct((1, D), jnp.float32)
    else:
        body = _dense_body
        out_specs = pl.BlockSpec((R, D), blk)
        out_shape = jax.ShapeDtypeStruct((N, D), jnp.float32)
    return pl.pallas_call(
        body, grid=grid, in_specs=in_specs, out_specs=out_specs,
        out_shape=out_shape,
    )(p0, p1, s_col, W, b, fW, fb, g, be)


def kernel(h, edge_index, edge_weight, W1, b1, W2, b2, fcW1, fcb1, fcW2,
           fcb2, bn1_gamma, bn1_beta, bn2_gamma, bn2_beta):
    N, D = h.shape
    E = edge_index.shape[1]
    Np = -(-N // 256) * 256

    src = edge_index[0]
    dst = edge_index[1]

    sout_p, sin_p = _make_deg_kernel(N, E, Np)(src, dst)

    K = 288
    epw = -(-(E // NW) // K) * K
    Ep = epw * NW
    src_p = jnp.pad(src, (0, Ep - E))
    dst_p = jnp.pad(dst, (0, Ep - E))
    ew_p = jnp.pad(edge_weight, (0, Ep - E))
    agg = _make_agg_kernel(N, Ep, D, Np)

    s_in = sin_p[:N, None]
    a1 = agg(h, src_p, dst_p, ew_p, sout_p)
    x1 = _tc_dense(a1[0], a1[1], s_in, W1, b1[None, :], fcW1, fcb1[None, :],
                   bn1_gamma[None, :], bn1_beta[None, :], mean=False)
    a2 = agg(x1, src_p, dst_p, ew_p, sout_p)
    out = _tc_dense(a2[0], a2[1], s_in, W2, b2[None, :], fcW2, fcb2[None, :],
                    bn2_gamma[None, :], bn2_beta[None, :], mean=True)
    return out


# speedup vs baseline: 6.6160x; 1.5494x over previous
import functools

import jax
import jax.numpy as jnp
from jax import lax
from jax.experimental import pallas as pl
from jax.experimental.pallas import tpu as pltpu
from jax.experimental.pallas import tpu_sc as plsc

NC = 2
NS = 16
NW = NC * NS


def _rsqrt16(x):
    i = lax.bitcast_convert_type(x, jnp.int32)
    i = jnp.int32(0x5F3759DF) - lax.shift_right_logical(i, 1)
    y = lax.bitcast_convert_type(i, jnp.float32)
    for _ in range(3):
        y = y * (1.5 - 0.5 * x * y * y)
    return y


@functools.lru_cache(maxsize=None)
def _make_deg_kernel(N, E, Np):
    per_tile = E // NS
    CH = 2000
    assert per_tile % CH == 0
    nwin = per_tile // CH
    SL = Np // NS
    assert SL % 16 == 0 and SL % 8 == 0
    mesh = plsc.VectorSubcoreMesh(core_axis_name="c", subcore_axis_name="s")

    @functools.partial(
        pl.kernel,
        out_type=(jax.ShapeDtypeStruct((Np,), jnp.float32),
                  jax.ShapeDtypeStruct((Np,), jnp.float32)),
        mesh=mesh,
        scratch_types=[
            pltpu.VMEM((CH,), jnp.int32),
            pltpu.VMEM((CH,), jnp.float32),
            pltpu.VMEM((SL,), jnp.float32),
            pltpu.VMEM_SHARED((Np,), jnp.float32),
        ],
        compiler_params=pltpu.CompilerParams(needs_layout_passes=False),
    )
    def deg_k(src_hbm, dst_hbm, sout_hbm, sin_hbm, idx_v, ones_v, slc_v,
              cnt_sh):
        c = lax.axis_index("c")
        s = lax.axis_index("s")
        for j in range(SL // 16):
            slc_v[pl.ds(16 * j, 16)] = jnp.zeros((16,), jnp.float32)
        pltpu.sync_copy(slc_v, cnt_sh.at[pl.ds(s * SL, SL)])
        for j in range(CH // 16):
            ones_v[pl.ds(16 * j, 16)] = jnp.ones((16,), jnp.float32)
        plsc.subcore_barrier()

        def win_body(w, carry):
            base = s * per_tile + w * CH

            @pl.when(c == 0)
            def _():
                pltpu.sync_copy(src_hbm.at[pl.ds(base, CH)], idx_v)

            @pl.when(c == 1)
            def _():
                pltpu.sync_copy(dst_hbm.at[pl.ds(base, CH)], idx_v)

            pltpu.sync_copy(ones_v, cnt_sh.at[idx_v], add=True)
            return carry

        lax.fori_loop(0, nwin, win_body, 0)
        plsc.subcore_barrier()

        pltpu.sync_copy(cnt_sh.at[pl.ds(s * SL, SL)], slc_v)
        for j in range(SL // 16):
            x = jnp.maximum(slc_v[pl.ds(16 * j, 16)], 1.0)
            slc_v[pl.ds(16 * j, 16)] = _rsqrt16(x)

        @pl.when(c == 0)
        def _():
            pltpu.sync_copy(slc_v, sout_hbm.at[pl.ds(s * SL, SL)])

        @pl.when(c == 1)
        def _():
            pltpu.sync_copy(slc_v, sin_hbm.at[pl.ds(s * SL, SL)])

    return deg_k


@functools.lru_cache(maxsize=None)
def _make_agg_kernel(N, E, D, Np):
    EPW = E // NW
    K = 144
    assert EPW % (2 * K) == 0
    NWIN = EPW // K
    RSTEP = (N // NS) & ~7
    RPT = N - RSTEP * (NS - 1)
    assert RPT % 8 == 0 and RPT >= RSTEP
    mesh = plsc.VectorSubcoreMesh(core_axis_name="c", subcore_axis_name="s")

    @functools.partial(
        pl.kernel,
        out_type=jax.ShapeDtypeStruct((NC, N, D), jnp.float32),
        mesh=mesh,
        scratch_types=[
            pltpu.VMEM((K,), jnp.int32),
            pltpu.VMEM((K,), jnp.int32),
            pltpu.VMEM((K,), jnp.float32),
            pltpu.VMEM((K, D), jnp.float32),
            pltpu.VMEM((K,), jnp.int32),
            pltpu.VMEM((K,), jnp.int32),
            pltpu.VMEM((K,), jnp.float32),
            pltpu.VMEM((K, D), jnp.float32),
            pltpu.VMEM((Np,), jnp.float32),
            pltpu.VMEM_SHARED((N, D), jnp.float32),
            pltpu.SemaphoreType.DMA,
            pltpu.SemaphoreType.DMA,
            pltpu.SemaphoreType.DMA,
            pltpu.SemaphoreType.DMA,
        ],
        compiler_params=pltpu.CompilerParams(needs_layout_passes=False),
    )
    def agg_k(x_hbm, src_hbm, dst_hbm, ew_hbm, sout_hbm, out_hbm,
              is_a, id_a, ew_a, rows_a, is_b, id_b, ew_b, rows_b,
              tab_v, agg_sh, gsem_a, gsem_b, ssem_a, ssem_b):
        c = lax.axis_index("c")
        s = lax.axis_index("s")
        wid = c * NS + s
        pltpu.sync_copy(sout_hbm, tab_v)

        def zr(i, carry):
            for j in range(D // 16):
                z = jnp.zeros((16,), jnp.float32)
                rows_a[i, pl.ds(16 * j, 16)] = z
                rows_b[i, pl.ds(16 * j, 16)] = z
            return carry

        lax.fori_loop(0, K, zr, 0)
        done = 0
        while done < RPT:
            step = min(K, RPT - done)
            pltpu.sync_copy(rows_a.at[pl.ds(0, step), :],
                            agg_sh.at[pl.ds(s * RSTEP + done, step), :])
            done += step
        plsc.subcore_barrier()

        def stage(e0, is_v, id_v, ew_v):
            pltpu.sync_copy(src_hbm.at[pl.ds(e0, K)], is_v)
            pltpu.sync_copy(dst_hbm.at[pl.ds(e0, K)], id_v)
            pltpu.sync_copy(ew_hbm.at[pl.ds(e0, K)], ew_v)

            def wcomp(i, c2):
                iv = is_v[pl.ds(i * 16, 16)]
                sov = plsc.load_gather(tab_v, [iv])
                ew_v[pl.ds(i * 16, 16)] = ew_v[pl.ds(i * 16, 16)] * sov
                return c2

            lax.fori_loop(0, K // 16, wcomp, 0)

        def scale(rows_v, ew_v):
            def body(e, c2):
                wv = plsc.load_gather(ew_v, [jnp.full((16,), e, jnp.int32)])
                for j in range(D // 16):
                    rows_v[e, pl.ds(16 * j, 16)] = (
                        rows_v[e, pl.ds(16 * j, 16)] * wv)
                return c2

            lax.fori_loop(0, K, body, 0)

        def pair_body(p, carry):
            e0 = wid * EPW + p * (2 * K)
            stage(e0, is_a, id_a, ew_a)
            ga = pltpu.async_copy(x_hbm.at[is_a], rows_a, gsem_a)
            stage(e0 + K, is_b, id_b, ew_b)
            gb = pltpu.async_copy(x_hbm.at[is_b], rows_b, gsem_b)
            ga.wait()
            scale(rows_a, ew_a)
            sa = pltpu.async_copy(rows_a, agg_sh.at[id_a], ssem_a, add=True)
            gb.wait()
            scale(rows_b, ew_b)
            sb = pltpu.async_copy(rows_b, agg_sh.at[id_b], ssem_b, add=True)
            sa.wait()
            sb.wait()
            return carry

        lax.fori_loop(0, NWIN // 2, pair_body, 0)
        plsc.subcore_barrier()
        pltpu.sync_copy(agg_sh.at[pl.ds(s * RSTEP, RPT), :],
                        out_hbm.at[c, pl.ds(s * RSTEP, RPT), :])

    return agg_k


_BN_INV = 0.9999950000374997


def _dense_body(p0_r, p1_r, s_r, W_r, b_r, fW_r, fb_r, g_r, be_r, o_r):
    a = (p0_r[...] + p1_r[...]) * s_r[...]
    t = jnp.dot(a, W_r[...], preferred_element_type=jnp.float32) + b_r[...]
    t = jnp.maximum(t, 0.0)
    t = jnp.dot(t, fW_r[...], preferred_element_type=jnp.float32) + fb_r[...]
    t = jnp.maximum(t, 0.0)
    o_r[...] = t * (g_r[...] * _BN_INV) + be_r[...]


def _dense_mean_body(nblocks, n_total, p0_r, p1_r, s_r, W_r, b_r, fW_r, fb_r,
                     g_r, be_r, o_r):
    i = pl.program_id(0)
    a = (p0_r[...] + p1_r[...]) * s_r[...]
    t = jnp.dot(a, W_r[...], preferred_element_type=jnp.float32) + b_r[...]
    t = jnp.maximum(t, 0.0)
    t = jnp.dot(t, fW_r[...], preferred_element_type=jnp.float32) + fb_r[...]
    t = jnp.maximum(t, 0.0)
    x = t * (g_r[...] * _BN_INV) + be_r[...]
    part = jnp.sum(x, axis=0, keepdims=True) * (1.0 / n_total)

    @pl.when(i == 0)
    def _():
        o_r[...] = part

    @pl.when(i > 0)
    def _():
        o_r[...] = o_r[...] + part


def _tc_dense(p0, p1, s_col, W, b, fW, fb, g, be, mean):
    N, D = p0.shape
    R = 1000
    assert N % R == 0
    grid = (N // R,)
    full = lambda i: (0, 0)
    blk = lambda i: (i, 0)
    in_specs = [
        pl.BlockSpec((R, D), blk),
        pl.BlockSpec((R, D), blk),
        pl.BlockSpec((R, 1), blk),
        pl.BlockSpec((D, D), full),
        pl.BlockSpec((1, D), full),
        pl.BlockSpec((D, D), full),
        pl.BlockSpec((1, D), full),
        pl.BlockSpec((1, D), full),
        pl.BlockSpec((1, D), full),
    ]
    if mean:
        body = functools.partial(_dense_mean_body, N // R, N)
        out_specs = pl.BlockSpec((1, D), full)
        out_shape = jax.ShapeDtypeStruct((1, D), jnp.float32)
    else:
        body = _dense_body
        out_specs = pl.BlockSpec((R, D), blk)
        out_shape = jax.ShapeDtypeStruct((N, D), jnp.float32)
    return pl.pallas_call(
        body, grid=grid, in_specs=in_specs, out_specs=out_specs,
        out_shape=out_shape,
    )(p0, p1, s_col, W, b, fW, fb, g, be)


def kernel(h, edge_index, edge_weight, W1, b1, W2, b2, fcW1, fcb1, fcW2,
           fcb2, bn1_gamma, bn1_beta, bn2_gamma, bn2_beta):
    N, D = h.shape
    E = edge_index.shape[1]
    Np = -(-N // 256) * 256

    src = edge_index[0]
    dst = edge_index[1]

    sout_p, sin_p = _make_deg_kernel(N, E, Np)(src, dst)

    WPAIR = 288
    epw = -(-(E // NW) // WPAIR) * WPAIR
    Ep = epw * NW
    pad_idx = (jnp.arange(Ep - E, dtype=jnp.int32) * 97) % N
    src_p = jnp.concatenate([src, pad_idx])
    dst_p = jnp.concatenate([dst, pad_idx])
    ew_p = jnp.pad(edge_weight, (0, Ep - E))
    agg = _make_agg_kernel(N, Ep, D, Np)

    s_in = sin_p[:N, None]
    a1 = agg(h, src_p, dst_p, ew_p, sout_p)
    x1 = _tc_dense(a1[0], a1[1], s_in, W1, b1[None, :], fcW1, fcb1[None, :],
                   bn1_gamma[None, :], bn1_beta[None, :], mean=False)
    a2 = agg(x1, src_p, dst_p, ew_p, sout_p)
    out = _tc_dense(a2[0], a2[1], s_in, W2, b2[None, :], fcW2, fcb2[None, :],
                    bn2_gamma[None, :], bn2_beta[None, :], mean=True)
    return out


# speedup vs baseline: 7.2748x; 1.0996x over previous
import functools

import jax
import jax.numpy as jnp
from jax import lax
from jax.experimental import pallas as pl
from jax.experimental.pallas import tpu as pltpu
from jax.experimental.pallas import tpu_sc as plsc

NC = 2
NS = 16
NW = NC * NS


def _rsqrt16(x):
    i = lax.bitcast_convert_type(x, jnp.int32)
    i = jnp.int32(0x5F3759DF) - lax.shift_right_logical(i, 1)
    y = lax.bitcast_convert_type(i, jnp.float32)
    for _ in range(3):
        y = y * (1.5 - 0.5 * x * y * y)
    return y


@functools.lru_cache(maxsize=None)
def _make_deg_kernel(N, E, Np):
    per_tile = E // NS
    CH = 2000
    assert per_tile % CH == 0
    nwin = per_tile // CH
    SL = Np // NS
    assert SL % 16 == 0 and SL % 8 == 0
    mesh = plsc.VectorSubcoreMesh(core_axis_name="c", subcore_axis_name="s")

    @functools.partial(
        pl.kernel,
        out_type=(jax.ShapeDtypeStruct((Np,), jnp.float32),
                  jax.ShapeDtypeStruct((Np,), jnp.float32)),
        mesh=mesh,
        scratch_types=[
            pltpu.VMEM((CH,), jnp.int32),
            pltpu.VMEM((CH,), jnp.float32),
            pltpu.VMEM((SL,), jnp.float32),
            pltpu.VMEM_SHARED((Np,), jnp.float32),
        ],
        compiler_params=pltpu.CompilerParams(needs_layout_passes=False),
    )
    def deg_k(src_hbm, dst_hbm, sout_hbm, sin_hbm, idx_v, ones_v, slc_v,
              cnt_sh):
        c = lax.axis_index("c")
        s = lax.axis_index("s")
        for j in range(SL // 16):
            slc_v[pl.ds(16 * j, 16)] = jnp.zeros((16,), jnp.float32)
        pltpu.sync_copy(slc_v, cnt_sh.at[pl.ds(s * SL, SL)])
        for j in range(CH // 16):
            ones_v[pl.ds(16 * j, 16)] = jnp.ones((16,), jnp.float32)
        plsc.subcore_barrier()

        def win_body(w, carry):
            base = s * per_tile + w * CH

            @pl.when(c == 0)
            def _():
                pltpu.sync_copy(src_hbm.at[pl.ds(base, CH)], idx_v)

            @pl.when(c == 1)
            def _():
                pltpu.sync_copy(dst_hbm.at[pl.ds(base, CH)], idx_v)

            pltpu.sync_copy(ones_v, cnt_sh.at[idx_v], add=True)
            return carry

        lax.fori_loop(0, nwin, win_body, 0)
        plsc.subcore_barrier()

        pltpu.sync_copy(cnt_sh.at[pl.ds(s * SL, SL)], slc_v)
        for j in range(SL // 16):
            x = jnp.maximum(slc_v[pl.ds(16 * j, 16)], 1.0)
            slc_v[pl.ds(16 * j, 16)] = _rsqrt16(x)

        @pl.when(c == 0)
        def _():
            pltpu.sync_copy(slc_v, sout_hbm.at[pl.ds(s * SL, SL)])

        @pl.when(c == 1)
        def _():
            pltpu.sync_copy(slc_v, sin_hbm.at[pl.ds(s * SL, SL)])

    return deg_k


@functools.lru_cache(maxsize=None)
def _make_agg_kernel(N, E, D, Np):
    EPW = E // NW
    K = 144
    assert EPW % (2 * K) == 0
    NWIN = EPW // K
    RSTEP = (N // NS) & ~7
    RPT = N - RSTEP * (NS - 1)
    assert RPT % 8 == 0 and RPT >= RSTEP
    mesh = plsc.VectorSubcoreMesh(core_axis_name="c", subcore_axis_name="s")

    @functools.partial(
        pl.kernel,
        out_type=jax.ShapeDtypeStruct((NC, N, D), jnp.float32),
        mesh=mesh,
        scratch_types=[
            pltpu.VMEM((K,), jnp.int32),
            pltpu.VMEM((K,), jnp.int32),
            pltpu.VMEM((K,), jnp.float32),
            pltpu.VMEM((K, D), jnp.float32),
            pltpu.VMEM((K,), jnp.int32),
            pltpu.VMEM((K,), jnp.int32),
            pltpu.VMEM((K,), jnp.float32),
            pltpu.VMEM((K, D), jnp.float32),
            pltpu.VMEM((Np,), jnp.float32),
            pltpu.VMEM_SHARED((N, D), jnp.float32),
            pltpu.SemaphoreType.DMA,
            pltpu.SemaphoreType.DMA,
            pltpu.SemaphoreType.DMA,
            pltpu.SemaphoreType.DMA,
        ],
        compiler_params=pltpu.CompilerParams(needs_layout_passes=False),
    )
    def agg_k(x_hbm, src_hbm, dst_hbm, ew_hbm, sout_hbm, out_hbm,
              is_a, id_a, ew_a, rows_a, is_b, id_b, ew_b, rows_b,
              tab_v, agg_sh, gsem_a, gsem_b, ssem_a, ssem_b):
        c = lax.axis_index("c")
        s = lax.axis_index("s")
        wid = c * NS + s
        pltpu.sync_copy(sout_hbm, tab_v)

        def zr(i, carry):
            for j in range(D // 16):
                z = jnp.zeros((16,), jnp.float32)
                rows_a[i, pl.ds(16 * j, 16)] = z
                rows_b[i, pl.ds(16 * j, 16)] = z
            return carry

        lax.fori_loop(0, K, zr, 0)
        done = 0
        while done < RPT:
            step = min(K, RPT - done)
            pltpu.sync_copy(rows_a.at[pl.ds(0, step), :],
                            agg_sh.at[pl.ds(s * RSTEP + done, step), :])
            done += step
        plsc.subcore_barrier()

        def stage(e0, is_v, id_v, ew_v):
            pltpu.sync_copy(src_hbm.at[pl.ds(e0, K)], is_v)
            pltpu.sync_copy(dst_hbm.at[pl.ds(e0, K)], id_v)
            pltpu.sync_copy(ew_hbm.at[pl.ds(e0, K)], ew_v)

            def wcomp(i, c2):
                iv = is_v[pl.ds(i * 16, 16)]
                sov = plsc.load_gather(tab_v, [iv])
                ew_v[pl.ds(i * 16, 16)] = ew_v[pl.ds(i * 16, 16)] * sov
                return c2

            lax.fori_loop(0, K // 16, wcomp, 0)

        def scale(rows_v, ew_v):
            def body(g, c2):
                wv16 = ew_v[pl.ds(g * 16, 16)]
                e0 = g * 16
                for t in range(16):
                    wt = wv16[t]
                    for j in range(D // 16):
                        rows_v[e0 + t, pl.ds(16 * j, 16)] = (
                            rows_v[e0 + t, pl.ds(16 * j, 16)] * wt)
                return c2

            lax.fori_loop(0, K // 16, body, 0)

        def pair_body(p, carry):
            e0 = wid * EPW + p * (2 * K)
            stage(e0, is_a, id_a, ew_a)
            ga = pltpu.async_copy(x_hbm.at[is_a], rows_a, gsem_a)
            stage(e0 + K, is_b, id_b, ew_b)
            gb = pltpu.async_copy(x_hbm.at[is_b], rows_b, gsem_b)
            ga.wait()
            scale(rows_a, ew_a)
            sa = pltpu.async_copy(rows_a, agg_sh.at[id_a], ssem_a, add=True)
            gb.wait()
            scale(rows_b, ew_b)
            sb = pltpu.async_copy(rows_b, agg_sh.at[id_b], ssem_b, add=True)
            sa.wait()
            sb.wait()
            return carry

        lax.fori_loop(0, NWIN // 2, pair_body, 0)
        plsc.subcore_barrier()
        pltpu.sync_copy(agg_sh.at[pl.ds(s * RSTEP, RPT), :],
                        out_hbm.at[c, pl.ds(s * RSTEP, RPT), :])

    return agg_k


_BN_INV = 0.9999950000374997


def _dense_body(p0_r, p1_r, s_r, W_r, b_r, fW_r, fb_r, g_r, be_r, o_r):
    a = (p0_r[...] + p1_r[...]) * s_r[...]
    t = jnp.dot(a, W_r[...], preferred_element_type=jnp.float32) + b_r[...]
    t = jnp.maximum(t, 0.0)
    t = jnp.dot(t, fW_r[...], preferred_element_type=jnp.float32) + fb_r[...]
    t = jnp.maximum(t, 0.0)
    o_r[...] = t * (g_r[...] * _BN_INV) + be_r[...]


def _dense_mean_body(nblocks, n_total, p0_r, p1_r, s_r, W_r, b_r, fW_r, fb_r,
                     g_r, be_r, o_r):
    i = pl.program_id(0)
    a = (p0_r[...] + p1_r[...]) * s_r[...]
    t = jnp.dot(a, W_r[...], preferred_element_type=jnp.float32) + b_r[...]
    t = jnp.maximum(t, 0.0)
    t = jnp.dot(t, fW_r[...], preferred_element_type=jnp.float32) + fb_r[...]
    t = jnp.maximum(t, 0.0)
    x = t * (g_r[...] * _BN_INV) + be_r[...]
    part = jnp.sum(x, axis=0, keepdims=True) * (1.0 / n_total)

    @pl.when(i == 0)
    def _():
        o_r[...] = part

    @pl.when(i > 0)
    def _():
        o_r[...] = o_r[...] + part


def _tc_dense(p0, p1, s_col, W, b, fW, fb, g, be, mean):
    N, D = p0.shape
    R = 1000
    assert N % R == 0
    grid = (N // R,)
    full = lambda i: (0, 0)
    blk = lambda i: (i, 0)
    in_specs = [
        pl.BlockSpec((R, D), blk),
        pl.BlockSpec((R, D), blk),
        pl.BlockSpec((R, 1), blk),
        pl.BlockSpec((D, D), full),
        pl.BlockSpec((1, D), full),
        pl.BlockSpec((D, D), full),
        pl.BlockSpec((1, D), full),
        pl.BlockSpec((1, D), full),
        pl.BlockSpec((1, D), full),
    ]
    if mean:
        body = functools.partial(_dense_mean_body, N // R, N)
        out_specs = pl.BlockSpec((1, D), full)
        out_shape = jax.ShapeDtypeStruct((1, D), jnp.float32)
    else:
        body = _dense_body
        out_specs = pl.BlockSpec((R, D), blk)
        out_shape = jax.ShapeDtypeStruct((N, D), jnp.float32)
    return pl.pallas_call(
        body, grid=grid, in_specs=in_specs, out_specs=out_specs,
        out_shape=out_shape,
    )(p0, p1, s_col, W, b, fW, fb, g, be)


def kernel(h, edge_index, edge_weight, W1, b1, W2, b2, fcW1, fcb1, fcW2,
           fcb2, bn1_gamma, bn1_beta, bn2_gamma, bn2_beta):
    N, D = h.shape
    E = edge_index.shape[1]
    Np = -(-N // 256) * 256

    src = edge_index[0]
    dst = edge_index[1]

    sout_p, sin_p = _make_deg_kernel(N, E, Np)(src, dst)

    WPAIR = 288
    epw = -(-(E // NW) // WPAIR) * WPAIR
    Ep = epw * NW
    pad_idx = (jnp.arange(Ep - E, dtype=jnp.int32) * 97) % N
    src_p = jnp.concatenate([src, pad_idx])
    dst_p = jnp.concatenate([dst, pad_idx])
    ew_p = jnp.pad(edge_weight, (0, Ep - E))
    agg = _make_agg_kernel(N, Ep, D, Np)

    s_in = sin_p[:N, None]
    a1 = agg(h, src_p, dst_p, ew_p, sout_p)
    x1 = _tc_dense(a1[0], a1[1], s_in, W1, b1[None, :], fcW1, fcb1[None, :],
                   bn1_gamma[None, :], bn1_beta[None, :], mean=False)
    a2 = agg(x1, src_p, dst_p, ew_p, sout_p)
    out = _tc_dense(a2[0], a2[1], s_in, W2, b2[None, :], fcW2, fcb2[None, :],
                    bn2_gamma[None, :], bn2_beta[None, :], mean=True)
    return out


# speedup vs baseline: 8.4963x; 1.1679x over previous
import functools

import jax
import jax.numpy as jnp
from jax import lax
from jax.experimental import pallas as pl
from jax.experimental.pallas import tpu as pltpu
from jax.experimental.pallas import tpu_sc as plsc

NC = 2
NS = 16
NW = NC * NS


def _rsqrt16(x):
    i = lax.bitcast_convert_type(x, jnp.int32)
    i = jnp.int32(0x5F3759DF) - lax.shift_right_logical(i, 1)
    y = lax.bitcast_convert_type(i, jnp.float32)
    for _ in range(3):
        y = y * (1.5 - 0.5 * x * y * y)
    return y


@functools.lru_cache(maxsize=None)
def _make_deg_kernel(N, E, Np):
    per_tile = E // NS
    CH = 2000
    assert per_tile % CH == 0
    nwin = per_tile // CH
    SL = Np // NS
    assert SL % 16 == 0 and SL % 8 == 0
    mesh = plsc.VectorSubcoreMesh(core_axis_name="c", subcore_axis_name="s")

    @functools.partial(
        pl.kernel,
        out_type=(jax.ShapeDtypeStruct((Np,), jnp.float32),
                  jax.ShapeDtypeStruct((Np,), jnp.float32)),
        mesh=mesh,
        scratch_types=[
            pltpu.VMEM((CH,), jnp.int32),
            pltpu.VMEM((CH,), jnp.float32),
            pltpu.VMEM((SL,), jnp.float32),
            pltpu.VMEM_SHARED((Np,), jnp.float32),
        ],
        compiler_params=pltpu.CompilerParams(needs_layout_passes=False),
    )
    def deg_k(src_hbm, dst_hbm, sout_hbm, sin_hbm, idx_v, ones_v, slc_v,
              cnt_sh):
        c = lax.axis_index("c")
        s = lax.axis_index("s")
        for j in range(SL // 16):
            slc_v[pl.ds(16 * j, 16)] = jnp.zeros((16,), jnp.float32)
        pltpu.sync_copy(slc_v, cnt_sh.at[pl.ds(s * SL, SL)])
        for j in range(CH // 16):
            ones_v[pl.ds(16 * j, 16)] = jnp.ones((16,), jnp.float32)
        plsc.subcore_barrier()

        def win_body(w, carry):
            base = s * per_tile + w * CH

            @pl.when(c == 0)
            def _():
                pltpu.sync_copy(src_hbm.at[pl.ds(base, CH)], idx_v)

            @pl.when(c == 1)
            def _():
                pltpu.sync_copy(dst_hbm.at[pl.ds(base, CH)], idx_v)

            pltpu.sync_copy(ones_v, cnt_sh.at[idx_v], add=True)
            return carry

        lax.fori_loop(0, nwin, win_body, 0)
        plsc.subcore_barrier()

        pltpu.sync_copy(cnt_sh.at[pl.ds(s * SL, SL)], slc_v)
        for j in range(SL // 16):
            x = jnp.maximum(slc_v[pl.ds(16 * j, 16)], 1.0)
            slc_v[pl.ds(16 * j, 16)] = _rsqrt16(x)

        @pl.when(c == 0)
        def _():
            pltpu.sync_copy(slc_v, sout_hbm.at[pl.ds(s * SL, SL)])

        @pl.when(c == 1)
        def _():
            pltpu.sync_copy(slc_v, sin_hbm.at[pl.ds(s * SL, SL)])

    return deg_k


@functools.lru_cache(maxsize=None)
def _make_agg_kernel(N, E, D, Np):
    EPW = E // NW
    K = 144
    assert EPW % (4 * K) == 0
    NQ = EPW // (4 * K)
    RSTEP = (N // NS) & ~7
    RPT = N - RSTEP * (NS - 1)
    assert RPT % 8 == 0 and RPT >= RSTEP
    mesh = plsc.VectorSubcoreMesh(core_axis_name="c", subcore_axis_name="s")

    @functools.partial(
        pl.kernel,
        out_type=jax.ShapeDtypeStruct((NC, N, D), jnp.float32),
        mesh=mesh,
        scratch_types=[
            pltpu.VMEM((K,), jnp.int32),
            pltpu.VMEM((K,), jnp.int32),
            pltpu.VMEM((K,), jnp.int32),
            pltpu.VMEM((K,), jnp.int32),
            pltpu.VMEM((K,), jnp.int32),
            pltpu.VMEM((K,), jnp.int32),
            pltpu.VMEM((K,), jnp.int32),
            pltpu.VMEM((K,), jnp.int32),
            pltpu.VMEM((K,), jnp.float32),
            pltpu.VMEM((K,), jnp.float32),
            pltpu.VMEM((K,), jnp.float32),
            pltpu.VMEM((K,), jnp.float32),
            pltpu.VMEM((K, D), jnp.float32),
            pltpu.VMEM((K, D), jnp.float32),
            pltpu.VMEM((Np,), jnp.float32),
            pltpu.VMEM_SHARED((N, D), jnp.float32),
            pltpu.SemaphoreType.DMA,
            pltpu.SemaphoreType.DMA,
            pltpu.SemaphoreType.DMA,
            pltpu.SemaphoreType.DMA,
            pltpu.SemaphoreType.DMA,
            pltpu.SemaphoreType.DMA,
        ],
        compiler_params=pltpu.CompilerParams(needs_layout_passes=False),
    )
    def agg_k(x_hbm, src_hbm, dst_hbm, ew_hbm, sout_hbm, out_hbm,
              is00, is01, is10, is11, id00, id01, id10, id11,
              ew00, ew01, ew10, ew11, rows_a, rows_b, tab_v, agg_sh,
              gsem_a, gsem_b, ssem_a, ssem_b, isem0, isem1):
        c = lax.axis_index("c")
        s = lax.axis_index("s")
        wid = c * NS + s
        isems = (isem0, isem1)
        is_t = ((is00, is01), (is10, is11))
        id_t = ((id00, id01), (id10, id11))
        ew_t = ((ew00, ew01), (ew10, ew11))
        pltpu.sync_copy(sout_hbm, tab_v)

        def zr(i, carry):
            for j in range(D // 16):
                z = jnp.zeros((16,), jnp.float32)
                rows_a[i, pl.ds(16 * j, 16)] = z
                rows_b[i, pl.ds(16 * j, 16)] = z
            return carry

        lax.fori_loop(0, K, zr, 0)
        done = 0
        while done < RPT:
            step = min(K, RPT - done)
            pltpu.sync_copy(rows_a.at[pl.ds(0, step), :],
                            agg_sh.at[pl.ds(s * RSTEP + done, step), :])
            done += step
        plsc.subcore_barrier()

        def fetch(par, p):
            e0 = wid * EPW + p * (2 * K)
            for win in range(2):
                d = pl.ds(e0 + win * K, K)
                pltpu.async_copy(src_hbm.at[d], is_t[par][win], isems[par])
                pltpu.async_copy(dst_hbm.at[d], id_t[par][win], isems[par])
                pltpu.async_copy(ew_hbm.at[d], ew_t[par][win], isems[par])

        def drain(par, p):
            e0 = wid * EPW + p * (2 * K)
            for win in range(2):
                d = pl.ds(e0 + win * K, K)
                pltpu.make_async_copy(src_hbm.at[d], is_t[par][win],
                                      isems[par]).wait()
                pltpu.make_async_copy(dst_hbm.at[d], id_t[par][win],
                                      isems[par]).wait()
                pltpu.make_async_copy(ew_hbm.at[d], ew_t[par][win],
                                      isems[par]).wait()

        def wcomp(par, win):
            is_v = is_t[par][win]
            ew_v = ew_t[par][win]

            def body(i, c2):
                iv = is_v[pl.ds(i * 16, 16)]
                sov = plsc.load_gather(tab_v, [iv])
                ew_v[pl.ds(i * 16, 16)] = ew_v[pl.ds(i * 16, 16)] * sov
                return c2

            lax.fori_loop(0, K // 16, body, 0)

        def scale(rows_v, par, win):
            ew_v = ew_t[par][win]

            def body(g, c2):
                wv16 = ew_v[pl.ds(g * 16, 16)]
                e0 = g * 16
                for t in range(16):
                    wt = wv16[t]
                    for j in range(D // 16):
                        rows_v[e0 + t, pl.ds(16 * j, 16)] = (
                            rows_v[e0 + t, pl.ds(16 * j, 16)] * wt)
                return c2

            lax.fori_loop(0, K // 16, body, 0)

        def process(par, p):
            wcomp(par, 0)
            ga = pltpu.async_copy(x_hbm.at[is_t[par][0]], rows_a, gsem_a)
            wcomp(par, 1)
            gb = pltpu.async_copy(x_hbm.at[is_t[par][1]], rows_b, gsem_b)
            ga.wait()
            scale(rows_a, par, 0)
            sa = pltpu.async_copy(rows_a, agg_sh.at[id_t[par][0]],
                                  ssem_a, add=True)
            gb.wait()
            scale(rows_b, par, 1)
            sb = pltpu.async_copy(rows_b, agg_sh.at[id_t[par][1]],
                                  ssem_b, add=True)
            sa.wait()
            sb.wait()

        fetch(0, 0)

        def sp_body(q, carry):
            p0 = 2 * q
            drain(0, p0)
            fetch(1, p0 + 1)
            process(0, p0)
            drain(1, p0 + 1)
            fetch(0, p0 + 2)
            process(1, p0 + 1)
            return carry

        lax.fori_loop(0, NQ, sp_body, 0)
        drain(0, 2 * NQ)
        plsc.subcore_barrier()
        pltpu.sync_copy(agg_sh.at[pl.ds(s * RSTEP, RPT), :],
                        out_hbm.at[c, pl.ds(s * RSTEP, RPT), :])

    return agg_k


_BN_INV = 0.9999950000374997


def _dense_body(p0_r, p1_r, s_r, W_r, b_r, fW_r, fb_r, g_r, be_r, o_r):
    a = (p0_r[...] + p1_r[...]) * s_r[...]
    t = jnp.dot(a, W_r[...], preferred_element_type=jnp.float32) + b_r[...]
    t = jnp.maximum(t, 0.0)
    t = jnp.dot(t, fW_r[...], preferred_element_type=jnp.float32) + fb_r[...]
    t = jnp.maximum(t, 0.0)
    o_r[...] = t * (g_r[...] * _BN_INV) + be_r[...]


def _dense_mean_body(nblocks, n_total, p0_r, p1_r, s_r, W_r, b_r, fW_r, fb_r,
                     g_r, be_r, o_r):
    i = pl.program_id(0)
    a = (p0_r[...] + p1_r[...]) * s_r[...]
    t = jnp.dot(a, W_r[...], preferred_element_type=jnp.float32) + b_r[...]
    t = jnp.maximum(t, 0.0)
    t = jnp.dot(t, fW_r[...], preferred_element_type=jnp.float32) + fb_r[...]
    t = jnp.maximum(t, 0.0)
    x = t * (g_r[...] * _BN_INV) + be_r[...]
    part = jnp.sum(x, axis=0, keepdims=True) * (1.0 / n_total)

    @pl.when(i == 0)
    def _():
        o_r[...] = part

    @pl.when(i > 0)
    def _():
        o_r[...] = o_r[...] + part


def _tc_dense(p0, p1, s_col, W, b, fW, fb, g, be, mean):
    N, D = p0.shape
    R = 1000
    assert N % R == 0
    grid = (N // R,)
    full = lambda i: (0, 0)
    blk = lambda i: (i, 0)
    in_specs = [
        pl.BlockSpec((R, D), blk),
        pl.BlockSpec((R, D), blk),
        pl.BlockSpec((R, 1), blk),
        pl.BlockSpec((D, D), full),
        pl.BlockSpec((1, D), full),
        pl.BlockSpec((D, D), full),
        pl.BlockSpec((1, D), full),
        pl.BlockSpec((1, D), full),
        pl.BlockSpec((1, D), full),
    ]
    if mean:
        body = functools.partial(_dense_mean_body, N // R, N)
        out_specs = pl.BlockSpec((1, D), full)
        out_shape = jax.ShapeDtypeStruct((1, D), jnp.float32)
    else:
        body = _dense_body
        out_specs = pl.BlockSpec((R, D), blk)
        out_shape = jax.ShapeDtypeStruct((N, D), jnp.float32)
    return pl.pallas_call(
        body, grid=grid, in_specs=in_specs, out_specs=out_specs,
        out_shape=out_shape,
    )(p0, p1, s_col, W, b, fW, fb, g, be)


def kernel(h, edge_index, edge_weight, W1, b1, W2, b2, fcW1, fcb1, fcW2,
           fcb2, bn1_gamma, bn1_beta, bn2_gamma, bn2_beta):
    N, D = h.shape
    E = edge_index.shape[1]
    Np = -(-N // 256) * 256

    src = edge_index[0]
    dst = edge_index[1]

    sout_p, sin_p = _make_deg_kernel(N, E, Np)(src, dst)

    WPAIR = 576
    epw = -(-(E // NW) // WPAIR) * WPAIR
    Ep = epw * NW
    pad_idx = (jnp.arange(Ep + 288 - E, dtype=jnp.int32) * 97) % N
    src_p = jnp.concatenate([src, pad_idx])
    dst_p = jnp.concatenate([dst, pad_idx])
    ew_p = jnp.pad(edge_weight, (0, Ep + 288 - E))
    agg = _make_agg_kernel(N, Ep, D, Np)

    s_in = sin_p[:N, None]
    a1 = agg(h, src_p, dst_p, ew_p, sout_p)
    x1 = _tc_dense(a1[0], a1[1], s_in, W1, b1[None, :], fcW1, fcb1[None, :],
                   bn1_gamma[None, :], bn1_beta[None, :], mean=False)
    a2 = agg(x1, src_p, dst_p, ew_p, sout_p)
    out = _tc_dense(a2[0], a2[1], s_in, W2, b2[None, :], fcW2, fcb2[None, :],
                    bn2_gamma[None, :], bn2_beta[None, :], mean=True)
    return out
